# baseline (speedup 1.0000x reference)
import functools

import jax
import jax.numpy as jnp
from jax import lax
from jax.experimental import pallas as pl
from jax.experimental.pallas import tpu as pltpu
from jax.experimental.pallas import tpu_sc as plsc

N = 10000
E = 320000
D = 128
L = 16

NC = 2
NS = 16
NW = NC * NS
CHUNK = 128
K = 8
E2 = 2 * E
CPW = 160
TPW = 2 * CPW
NB = TPW // K
EPAD = NW * CPW * CHUNK
ACC_N = 10112

_BM = 1000


def _mm_body(x_ref, w_ref, o_ref):
    o_ref[...] = jnp.dot(x_ref[...], w_ref[...],
                         preferred_element_type=jnp.float32)


def _matmul(x, w):
    return pl.pallas_call(
        _mm_body,
        grid=(N // _BM,),
        in_specs=[pl.BlockSpec((_BM, D), lambda i: (i, 0)),
                  pl.BlockSpec((D, D), lambda i: (0, 0))],
        out_specs=pl.BlockSpec((_BM, D), lambda i: (i, 0)),
        out_shape=jax.ShapeDtypeStruct((N, D), jnp.float32),
    )(x, w)


def _final_body(x_ref, wl_ref, b_ref, p0_ref, p1_ref, o_ref):
    xl = lax.dot_general(x_ref[...], wl_ref[...],
                         (((1,), (1,)), ((), ())),
                         preferred_element_type=jnp.float32)
    o_ref[...] = jnp.maximum(xl + b_ref[...] + p0_ref[...] + p1_ref[...], 0.0)


def _final(x, w_lin, b_lin, p0, p1):
    return pl.pallas_call(
        _final_body,
        grid=(N // _BM,),
        in_specs=[pl.BlockSpec((_BM, D), lambda i: (i, 0)),
                  pl.BlockSpec((D, D), lambda i: (0, 0)),
                  pl.BlockSpec((1, D), lambda i: (0, 0)),
                  pl.BlockSpec((_BM, D), lambda i: (i, 0)),
                  pl.BlockSpec((_BM, D), lambda i: (i, 0))],
        out_specs=pl.BlockSpec((_BM, D), lambda i: (i, 0)),
        out_shape=jax.ShapeDtypeStruct((N, D), jnp.float32),
    )(x, w_lin, b_lin.reshape(1, D), p0, p1)


_sc_mesh = plsc.VectorSubcoreMesh(core_axis_name="c", subcore_axis_name="s")


@functools.partial(
    pl.kernel,
    out_type=jax.ShapeDtypeStruct((NC, ACC_N, D), jnp.float32),
    mesh=_sc_mesh,
    scratch_types=[
        pltpu.VMEM((K, CHUNK), jnp.int32),
        pltpu.VMEM((K, CHUNK), jnp.int32),
        pltpu.VMEM((CHUNK, D), jnp.float32),
        pltpu.VMEM((CHUNK, D), jnp.float32),
        pltpu.VMEM_SHARED((ACC_N, D), jnp.float32),
        pltpu.SemaphoreType.DMA,
        pltpu.SemaphoreType.DMA,
        pltpu.SemaphoreType.DMA,
        pltpu.SemaphoreType.DMA,
    ],
)
def _sc_scatter(table_hbm, gsrc_hbm, dst_hbm, zeros_hbm, out_hbm,
                gidx_v, didx_v, rows_a, rows_b, acc,
                sga, sgb, ssa, ssb):
    c = lax.axis_index("c")
    s = lax.axis_index("s")
    gwid = c * NS + s

    zrows = ACC_N // NS
    pltpu.sync_copy(zeros_hbm.at[pl.ds(s * zrows, zrows)],
                    acc.at[pl.ds(s * zrows, zrows)])
    plsc.subcore_barrier()

    rows = (rows_a, rows_b)
    g_sems = (sga, sgb)
    s_sems = (ssa, ssb)

    def block_body(kb, carry):
        pltpu.sync_copy(gsrc_hbm.at[gwid, pl.ds(kb * K, K)], gidx_v)
        pltpu.sync_copy(dst_hbm.at[gwid, pl.ds(kb * K, K)], didx_v)

        g_desc = [None, None]
        s_desc = [None, None]

        def issue_gather(j):
            p = j % 2
            g_desc[p] = pltpu.async_copy(
                table_hbm.at[gidx_v.at[j]], rows[p], g_sems[p])

        issue_gather(0)
        for j in range(K):
            p = j % 2
            if j + 1 < K:
                if j >= 1:
                    s_desc[1 - p].wait()
                issue_gather(j + 1)
            g_desc[p].wait()
            s_desc[p] = pltpu.async_copy(
                rows[p], acc.at[didx_v.at[j]], s_sems[p], add=True)
        s_desc[0].wait()
        s_desc[1].wait()
        return carry

    lax.fori_loop(0, NB, block_body, 0)

    plsc.subcore_barrier()

    pltpu.sync_copy(acc.at[pl.ds(s * zrows, zrows)],
                    out_hbm.at[c, pl.ds(s * zrows, zrows)])


def kernel(x, edge_index, edge_label, W_dir, b_lab, W_lin, b_lin):
    xw = _matmul(x, W_dir)
    table = jnp.concatenate(
        [xw, jnp.zeros((1, D), jnp.float32), b_lab], axis=0)

    src = jnp.concatenate([edge_index[0], edge_index[1]])
    dst = jnp.concatenate([edge_index[1], edge_index[0]])
    lab = jnp.concatenate([edge_label, edge_label]) + jnp.int32(N + 1)

    pad = EPAD - E2
    padn = jnp.full((pad,), N, jnp.int32)
    src_c = jnp.concatenate([src, padn]).reshape(NW, CPW, CHUNK)
    dst_c = jnp.concatenate([dst, padn]).reshape(NW, CPW, CHUNK)
    lab_c = jnp.concatenate([lab, padn]).reshape(NW, CPW, CHUNK)

    gsrc = jnp.stack([src_c, lab_c], axis=2).reshape(NW, TPW, CHUNK)
    dstd = jnp.stack([dst_c, dst_c], axis=2).reshape(NW, TPW, CHUNK)

    zeros = jnp.zeros((ACC_N, D), jnp.float32)

    partials = _sc_scatter(table, gsrc, dstd, zeros)
    return _final(x, W_lin, b_lin, partials[0], partials[1])


# speedup vs baseline: 2.4129x; 2.4129x over previous
import functools

import jax
import jax.numpy as jnp
from jax import lax
from jax.experimental import pallas as pl
from jax.experimental.pallas import tpu as pltpu
from jax.experimental.pallas import tpu_sc as plsc

N = 10000
E = 320000
D = 128
L = 16

NC = 2
NS = 16
NW = NC * NS
CHUNK = 128
K = 16
E2 = 2 * E
CPW = 160
NB = CPW // K
EPAD = NW * CPW * CHUNK
ACC_N = 10112

_BM = 1000


def _mm_body(x_ref, w_ref, o_ref):
    o_ref[...] = jnp.dot(x_ref[...], w_ref[...],
                         preferred_element_type=jnp.float32)


def _matmul(x, w):
    return pl.pallas_call(
        _mm_body,
        grid=(N // _BM,),
        in_specs=[pl.BlockSpec((_BM, D), lambda i: (i, 0)),
                  pl.BlockSpec((D, D), lambda i: (0, 0))],
        out_specs=pl.BlockSpec((_BM, D), lambda i: (i, 0)),
        out_shape=jax.ShapeDtypeStruct((N, D), jnp.float32),
    )(x, w)


def _final_body(x_ref, wl_ref, b_ref, p0_ref, p1_ref, o_ref):
    xl = lax.dot_general(x_ref[...], wl_ref[...],
                         (((1,), (1,)), ((), ())),
                         preferred_element_type=jnp.float32)
    o_ref[...] = jnp.maximum(xl + b_ref[...] + p0_ref[...] + p1_ref[...], 0.0)


def _final(x, w_lin, b_lin, p0, p1):
    return pl.pallas_call(
        _final_body,
        grid=(N // _BM,),
        in_specs=[pl.BlockSpec((_BM, D), lambda i: (i, 0)),
                  pl.BlockSpec((D, D), lambda i: (0, 0)),
                  pl.BlockSpec((1, D), lambda i: (0, 0)),
                  pl.BlockSpec((_BM, D), lambda i: (i, 0)),
                  pl.BlockSpec((_BM, D), lambda i: (i, 0))],
        out_specs=pl.BlockSpec((_BM, D), lambda i: (i, 0)),
        out_shape=jax.ShapeDtypeStruct((N, D), jnp.float32),
    )(x, w_lin, b_lin.reshape(1, D), p0, p1)


_sc_mesh = plsc.VectorSubcoreMesh(core_axis_name="c", subcore_axis_name="s")


@functools.partial(
    pl.kernel,
    out_type=jax.ShapeDtypeStruct((NC, ACC_N, D), jnp.float32),
    mesh=_sc_mesh,
    scratch_types=[
        pltpu.VMEM((K, CHUNK), jnp.int32),
        pltpu.VMEM((K, CHUNK), jnp.int32),
        pltpu.VMEM((K, CHUNK), jnp.int32),
        pltpu.VMEM((L * D,), jnp.float32),
        pltpu.VMEM((CHUNK, D), jnp.float32),
        pltpu.VMEM((CHUNK, D), jnp.float32),
        pltpu.VMEM_SHARED((ACC_N, D), jnp.float32),
        pltpu.SemaphoreType.DMA,
        pltpu.SemaphoreType.DMA,
    ],
)
def _sc_scatter(table_hbm, blab_hbm, src_hbm, lab_hbm, dst_hbm,
                zeros_hbm, out_hbm,
                gidx_v, lidx_v, didx_v, blab_v, rows_a, rows_b, acc,
                sga, sgb):
    c = lax.axis_index("c")
    s = lax.axis_index("s")
    gwid = c * NS + s

    pltpu.sync_copy(blab_hbm, blab_v)

    zrows = ACC_N // NS
    pltpu.sync_copy(zeros_hbm.at[pl.ds(s * zrows, zrows)],
                    acc.at[pl.ds(s * zrows, zrows)])
    plsc.subcore_barrier()

    rows = (rows_a, rows_b)
    g_sems = (sga, sgb)

    def add_bias(j, buf):
        def group_body(g, carry):
            labv = lidx_v[j, pl.ds(g * 16, 16)]
            for e16 in range(16):
                base = labv[e16] * D
                e = g * 16 + e16
                for j2 in range(D // 16):
                    bv = blab_v[pl.ds(base + j2 * 16, 16)]
                    buf[e, pl.ds(j2 * 16, 16)] += bv
            return carry

        lax.fori_loop(0, CHUNK // 16, group_body, 0)

    def block_body(kb, carry):
        pltpu.sync_copy(src_hbm.at[gwid, pl.ds(kb * K, K)], gidx_v)
        pltpu.sync_copy(lab_hbm.at[gwid, pl.ds(kb * K, K)], lidx_v)
        pltpu.sync_copy(dst_hbm.at[gwid, pl.ds(kb * K, K)], didx_v)

        pltpu.async_copy(table_hbm.at[gidx_v.at[0]], rows[0], g_sems[0])

        def pair_body(i, carry2):
            ja = 2 * i
            jb = 2 * i + 1
            pltpu.async_copy(table_hbm.at[gidx_v.at[jb]], rows[1],
                             g_sems[1])
            pltpu.make_async_copy(table_hbm.at[gidx_v.at[ja]], rows[0],
                                  g_sems[0]).wait()
            add_bias(ja, rows[0])
            pltpu.sync_copy(rows[0], acc.at[didx_v.at[ja]], add=True)

            @pl.when(i < K // 2 - 1)
            def _():
                pltpu.async_copy(table_hbm.at[gidx_v.at[jb + 1]],
                                 rows[0], g_sems[0])

            pltpu.make_async_copy(table_hbm.at[gidx_v.at[jb]], rows[1],
                                  g_sems[1]).wait()
            add_bias(jb, rows[1])
            pltpu.sync_copy(rows[1], acc.at[didx_v.at[jb]], add=True)
            return carry2

        lax.fori_loop(0, K // 2, pair_body, 0)
        return carry

    lax.fori_loop(0, NB, block_body, 0)

    plsc.subcore_barrier()

    pltpu.sync_copy(acc.at[pl.ds(s * zrows, zrows)],
                    out_hbm.at[c, pl.ds(s * zrows, zrows)])


def kernel(x, edge_index, edge_label, W_dir, b_lab, W_lin, b_lin):
    xw = _matmul(x, W_dir)
    table = jnp.concatenate([xw, jnp.zeros((1, D), jnp.float32)], axis=0)

    src = jnp.concatenate([edge_index[0], edge_index[1]])
    dst = jnp.concatenate([edge_index[1], edge_index[0]])
    lab = jnp.concatenate([edge_label, edge_label])

    pad = EPAD - E2
    padn = jnp.full((pad,), N, jnp.int32)
    padz = jnp.zeros((pad,), jnp.int32)
    src_full = jnp.concatenate([src, padn]).reshape(NW, CPW, CHUNK)
    dst_full = jnp.concatenate([dst, padn]).reshape(NW, CPW, CHUNK)
    lab_full = jnp.concatenate([lab, padz]).reshape(NW, CPW, CHUNK)

    zeros = jnp.zeros((ACC_N, D), jnp.float32)

    partials = _sc_scatter(table, b_lab.reshape(L * D), src_full,
                           lab_full, dst_full, zeros)
    return _final(x, W_lin, b_lin, partials[0], partials[1])


# speedup vs baseline: 3.8345x; 1.5892x over previous
import functools

import jax
import jax.numpy as jnp
from jax import lax
from jax.experimental import pallas as pl
from jax.experimental.pallas import tpu as pltpu
from jax.experimental.pallas import tpu_sc as plsc

N = 10000
E = 320000
D = 128
L = 16

NC = 2
NS = 16
NW = NC * NS
CHUNK = 128
K = 16
E2 = 2 * E
CPW = 160
NB = CPW // K
EPAD = NW * CPW * CHUNK
ACC_N = 10112

_BM = 1000


def _table_body(x_ref, w_ref, blab_ref, o_ref):
    xw = jnp.dot(x_ref[...], w_ref[...], preferred_element_type=jnp.float32)
    msg = xw[:, None, :] + blab_ref[...][None, :, :]
    o_ref[...] = msg.reshape(_BM * L, D)


def _build_table(x, w_dir, b_lab):
    return pl.pallas_call(
        _table_body,
        grid=(N // _BM,),
        in_specs=[pl.BlockSpec((_BM, D), lambda i: (i, 0)),
                  pl.BlockSpec((D, D), lambda i: (0, 0)),
                  pl.BlockSpec((L, D), lambda i: (0, 0))],
        out_specs=pl.BlockSpec((_BM * L, D), lambda i: (i, 0)),
        out_shape=jax.ShapeDtypeStruct((N * L, D), jnp.float32),
    )(x, w_dir, b_lab)


def _final_body(x_ref, wl_ref, b_ref, p0_ref, p1_ref, o_ref):
    xl = lax.dot_general(x_ref[...], wl_ref[...],
                         (((1,), (1,)), ((), ())),
                         preferred_element_type=jnp.float32)
    o_ref[...] = jnp.maximum(xl + b_ref[...] + p0_ref[...] + p1_ref[...], 0.0)


def _final(x, w_lin, b_lin, p0, p1):
    return pl.pallas_call(
        _final_body,
        grid=(N // _BM,),
        in_specs=[pl.BlockSpec((_BM, D), lambda i: (i, 0)),
                  pl.BlockSpec((D, D), lambda i: (0, 0)),
                  pl.BlockSpec((1, D), lambda i: (0, 0)),
                  pl.BlockSpec((_BM, D), lambda i: (i, 0)),
                  pl.BlockSpec((_BM, D), lambda i: (i, 0))],
        out_specs=pl.BlockSpec((_BM, D), lambda i: (i, 0)),
        out_shape=jax.ShapeDtypeStruct((N, D), jnp.float32),
    )(x, w_lin, b_lin.reshape(1, D), p0, p1)


_sc_mesh = plsc.VectorSubcoreMesh(core_axis_name="c", subcore_axis_name="s")


@functools.partial(
    pl.kernel,
    out_type=jax.ShapeDtypeStruct((NC, ACC_N, D), jnp.float32),
    mesh=_sc_mesh,
    scratch_types=[
        pltpu.VMEM((K, CHUNK), jnp.int32),
        pltpu.VMEM((K, CHUNK), jnp.int32),
        pltpu.VMEM((CHUNK, D), jnp.float32),
        pltpu.VMEM((CHUNK, D), jnp.float32),
        pltpu.VMEM_SHARED((ACC_N, D), jnp.float32),
        pltpu.SemaphoreType.DMA,
        pltpu.SemaphoreType.DMA,
    ],
)
def _sc_scatter(table_hbm, gsrc_hbm, dst_hbm, zeros_hbm, out_hbm,
                gidx_v, didx_v, rows_a, rows_b, acc, sga, sgb):
    c = lax.axis_index("c")
    s = lax.axis_index("s")
    gwid = c * NS + s

    zrows = ACC_N // NS
    pltpu.sync_copy(zeros_hbm.at[pl.ds(s * zrows, zrows)],
                    acc.at[pl.ds(s * zrows, zrows)])
    plsc.subcore_barrier()

    rows = (rows_a, rows_b)
    g_sems = (sga, sgb)

    def block_body(kb, carry):
        pltpu.sync_copy(gsrc_hbm.at[gwid, pl.ds(kb * K, K)], gidx_v)
        pltpu.sync_copy(dst_hbm.at[gwid, pl.ds(kb * K, K)], didx_v)

        desc = pltpu.async_copy(table_hbm.at[gidx_v.at[0]], rows[0],
                                g_sems[0])
        for j in range(K):
            p = j % 2
            if j + 1 < K:
                ndesc = pltpu.async_copy(
                    table_hbm.at[gidx_v.at[j + 1]], rows[1 - p],
                    g_sems[1 - p])
            desc.wait()
            pltpu.sync_copy(rows[p], acc.at[didx_v.at[j]], add=True)
            if j + 1 < K:
                desc = ndesc
        return carry

    lax.fori_loop(0, NB, block_body, 0)

    plsc.subcore_barrier()

    pltpu.sync_copy(acc.at[pl.ds(s * zrows, zrows)],
                    out_hbm.at[c, pl.ds(s * zrows, zrows)])


def kernel(x, edge_index, edge_label, W_dir, b_lab, W_lin, b_lin):
    table = _build_table(x, W_dir, b_lab)

    src = jnp.concatenate([edge_index[0], edge_index[1]])
    dst = jnp.concatenate([edge_index[1], edge_index[0]])
    lab = jnp.concatenate([edge_label, edge_label])
    gsrc = src * jnp.int32(L) + lab

    pad = EPAD - E2
    padz = jnp.zeros((pad,), jnp.int32)
    padn = jnp.full((pad,), N, jnp.int32)
    gsrc_full = jnp.concatenate([gsrc, padz]).reshape(NW, CPW, CHUNK)
    dst_full = jnp.concatenate([dst, padn]).reshape(NW, CPW, CHUNK)

    zeros = jnp.zeros((ACC_N, D), jnp.float32)

    partials = _sc_scatter(table, gsrc_full, dst_full, zeros)
    return _final(x, W_lin, b_lin, partials[0], partials[1])


# speedup vs baseline: 3.8348x; 1.0001x over previous
import functools

import jax
import jax.numpy as jnp
from jax import lax
from jax.experimental import pallas as pl
from jax.experimental.pallas import tpu as pltpu
from jax.experimental.pallas import tpu_sc as plsc

N = 10000
E = 320000
D = 128
L = 16

NC = 2
NS = 16
NW = NC * NS
CHUNK = 128
K = 16
E2 = 2 * E
CPW = 160
NB = CPW // K
EPAD = NW * CPW * CHUNK
ACC_N = 10112

_BM = 1000


def _table_body(x_ref, w_ref, blab_ref, o_ref):
    xw = jnp.dot(x_ref[...], w_ref[...], preferred_element_type=jnp.float32)
    msg = xw[:, None, :] + blab_ref[...][None, :, :]
    o_ref[...] = msg.reshape(_BM * L, D)


def _build_table(x, w_dir, b_lab):
    return pl.pallas_call(
        _table_body,
        grid=(N // _BM,),
        in_specs=[pl.BlockSpec((_BM, D), lambda i: (i, 0)),
                  pl.BlockSpec((D, D), lambda i: (0, 0)),
                  pl.BlockSpec((L, D), lambda i: (0, 0))],
        out_specs=pl.BlockSpec((_BM * L, D), lambda i: (i, 0)),
        out_shape=jax.ShapeDtypeStruct((N * L, D), jnp.float32),
    )(x, w_dir, b_lab)


def _final_body(x_ref, wl_ref, b_ref, p0_ref, p1_ref, o_ref):
    xl = lax.dot_general(x_ref[...], wl_ref[...],
                         (((1,), (1,)), ((), ())),
                         preferred_element_type=jnp.float32)
    o_ref[...] = jnp.maximum(xl + b_ref[...] + p0_ref[...] + p1_ref[...], 0.0)


def _final(x, w_lin, b_lin, p0, p1):
    return pl.pallas_call(
        _final_body,
        grid=(N // _BM,),
        in_specs=[pl.BlockSpec((_BM, D), lambda i: (i, 0)),
                  pl.BlockSpec((D, D), lambda i: (0, 0)),
                  pl.BlockSpec((1, D), lambda i: (0, 0)),
                  pl.BlockSpec((_BM, D), lambda i: (i, 0)),
                  pl.BlockSpec((_BM, D), lambda i: (i, 0))],
        out_specs=pl.BlockSpec((_BM, D), lambda i: (i, 0)),
        out_shape=jax.ShapeDtypeStruct((N, D), jnp.float32),
    )(x, w_lin, b_lin.reshape(1, D), p0, p1)


_sc_mesh = plsc.VectorSubcoreMesh(core_axis_name="c", subcore_axis_name="s")


@functools.partial(
    pl.kernel,
    out_type=jax.ShapeDtypeStruct((NC, ACC_N, D), jnp.float32),
    mesh=_sc_mesh,
    scratch_types=[
        pltpu.VMEM((K, CHUNK), jnp.int32),
        pltpu.VMEM((K, CHUNK), jnp.int32),
        pltpu.VMEM((CHUNK, D), jnp.float32),
        pltpu.VMEM((CHUNK, D), jnp.float32),
        pltpu.VMEM_SHARED((ACC_N, D), jnp.float32),
        pltpu.SemaphoreType.DMA,
        pltpu.SemaphoreType.DMA,
        pltpu.SemaphoreType.DMA,
        pltpu.SemaphoreType.DMA,
    ],
)
def _sc_scatter(table_hbm, gsrc_hbm, dst_hbm, zeros_hbm, out_hbm,
                gidx_v, didx_v, rows_a, rows_b, acc, sga, sgb, ssa, ssb):
    c = lax.axis_index("c")
    s = lax.axis_index("s")
    gwid = c * NS + s

    zrows = ACC_N // NS
    pltpu.sync_copy(zeros_hbm.at[pl.ds(s * zrows, zrows)],
                    acc.at[pl.ds(s * zrows, zrows)])
    plsc.subcore_barrier()

    rows = (rows_a, rows_b)
    g_sems = (sga, sgb)
    s_sems = (ssa, ssb)

    def block_body(kb, carry):
        pltpu.sync_copy(gsrc_hbm.at[gwid, pl.ds(kb * K, K)], gidx_v)
        pltpu.sync_copy(dst_hbm.at[gwid, pl.ds(kb * K, K)], didx_v)

        g_desc = [None, None]
        s_desc = [None, None]
        g_desc[0] = pltpu.async_copy(table_hbm.at[gidx_v.at[0]], rows[0],
                                     g_sems[0])
        for j in range(K):
            p = j % 2
            if j + 1 < K:
                if j >= 1:
                    s_desc[1 - p].wait()
                g_desc[1 - p] = pltpu.async_copy(
                    table_hbm.at[gidx_v.at[j + 1]], rows[1 - p],
                    g_sems[1 - p])
            g_desc[p].wait()
            s_desc[p] = pltpu.async_copy(
                rows[p], acc.at[didx_v.at[j]], s_sems[p], add=True)
        s_desc[0].wait()
        s_desc[1].wait()
        return carry

    lax.fori_loop(0, NB, block_body, 0)

    plsc.subcore_barrier()

    pltpu.sync_copy(acc.at[pl.ds(s * zrows, zrows)],
                    out_hbm.at[c, pl.ds(s * zrows, zrows)])


def kernel(x, edge_index, edge_label, W_dir, b_lab, W_lin, b_lin):
    table = _build_table(x, W_dir, b_lab)

    src = jnp.concatenate([edge_index[0], edge_index[1]])
    dst = jnp.concatenate([edge_index[1], edge_index[0]])
    lab = jnp.concatenate([edge_label, edge_label])
    gsrc = src * jnp.int32(L) + lab

    pad = EPAD - E2
    padz = jnp.zeros((pad,), jnp.int32)
    padn = jnp.full((pad,), N, jnp.int32)
    gsrc_full = jnp.concatenate([gsrc, padz]).reshape(NW, CPW, CHUNK)
    dst_full = jnp.concatenate([dst, padn]).reshape(NW, CPW, CHUNK)

    zeros = jnp.zeros((ACC_N, D), jnp.float32)

    partials = _sc_scatter(table, gsrc_full, dst_full, zeros)
    return _final(x, W_lin, b_lin, partials[0], partials[1])


# speedup vs baseline: 3.8835x; 1.0127x over previous
import functools

import jax
import jax.numpy as jnp
from jax import lax
from jax.experimental import pallas as pl
from jax.experimental.pallas import tpu as pltpu
from jax.experimental.pallas import tpu_sc as plsc

N = 10000
E = 320000
D = 128
L = 16

NC = 2
NS = 16
NW = NC * NS
CHUNK = 128
K = 32
E2 = 2 * E
CPW = 160
NB = CPW // K
EPAD = NW * CPW * CHUNK
ACC_N = 10112

_BM = 1000


def _table_body(x_ref, w_ref, blab_ref, o_ref):
    xw = jnp.dot(x_ref[...], w_ref[...], preferred_element_type=jnp.float32)
    msg = xw[:, None, :] + blab_ref[...][None, :, :]
    o_ref[...] = msg.reshape(_BM * L, D)


def _build_table(x, w_dir, b_lab):
    return pl.pallas_call(
        _table_body,
        grid=(N // _BM,),
        in_specs=[pl.BlockSpec((_BM, D), lambda i: (i, 0)),
                  pl.BlockSpec((D, D), lambda i: (0, 0)),
                  pl.BlockSpec((L, D), lambda i: (0, 0))],
        out_specs=pl.BlockSpec((_BM * L, D), lambda i: (i, 0)),
        out_shape=jax.ShapeDtypeStruct((N * L, D), jnp.float32),
    )(x, w_dir, b_lab)


def _final_body(x_ref, wl_ref, b_ref, p0_ref, p1_ref, o_ref):
    xl = lax.dot_general(x_ref[...], wl_ref[...],
                         (((1,), (1,)), ((), ())),
                         preferred_element_type=jnp.float32)
    o_ref[...] = jnp.maximum(xl + b_ref[...] + p0_ref[...] + p1_ref[...], 0.0)


def _final(x, w_lin, b_lin, p0, p1):
    return pl.pallas_call(
        _final_body,
        grid=(N // _BM,),
        in_specs=[pl.BlockSpec((_BM, D), lambda i: (i, 0)),
                  pl.BlockSpec((D, D), lambda i: (0, 0)),
                  pl.BlockSpec((1, D), lambda i: (0, 0)),
                  pl.BlockSpec((_BM, D), lambda i: (i, 0)),
                  pl.BlockSpec((_BM, D), lambda i: (i, 0))],
        out_specs=pl.BlockSpec((_BM, D), lambda i: (i, 0)),
        out_shape=jax.ShapeDtypeStruct((N, D), jnp.float32),
    )(x, w_lin, b_lin.reshape(1, D), p0, p1)


_sc_mesh = plsc.VectorSubcoreMesh(core_axis_name="c", subcore_axis_name="s")


@functools.partial(
    pl.kernel,
    out_type=jax.ShapeDtypeStruct((NC, ACC_N, D), jnp.float32),
    mesh=_sc_mesh,
    scratch_types=[
        pltpu.VMEM((K, CHUNK), jnp.int32),
        pltpu.VMEM((K, CHUNK), jnp.int32),
        pltpu.VMEM((CHUNK, D), jnp.float32),
        pltpu.VMEM((CHUNK, D), jnp.float32),
        pltpu.VMEM_SHARED((ACC_N, D), jnp.float32),
        pltpu.SemaphoreType.DMA,
        pltpu.SemaphoreType.DMA,
    ],
)
def _sc_scatter(table_hbm, gsrc_hbm, dst_hbm, zeros_hbm, out_hbm,
                gidx_v, didx_v, rows_a, rows_b, acc, sga, sgb):
    c = lax.axis_index("c")
    s = lax.axis_index("s")
    gwid = c * NS + s

    zrows = ACC_N // NS
    pltpu.sync_copy(zeros_hbm.at[pl.ds(s * zrows, zrows)],
                    acc.at[pl.ds(s * zrows, zrows)])
    plsc.subcore_barrier()

    rows = (rows_a, rows_b)
    g_sems = (sga, sgb)

    def block_body(kb, carry):
        pltpu.sync_copy(gsrc_hbm.at[gwid, pl.ds(kb * K, K)], gidx_v)
        pltpu.sync_copy(dst_hbm.at[gwid, pl.ds(kb * K, K)], didx_v)

        desc = pltpu.async_copy(table_hbm.at[gidx_v.at[0]], rows[0],
                                g_sems[0])
        for j in range(K):
            p = j % 2
            if j + 1 < K:
                ndesc = pltpu.async_copy(
                    table_hbm.at[gidx_v.at[j + 1]], rows[1 - p],
                    g_sems[1 - p])
            desc.wait()
            pltpu.sync_copy(rows[p], acc.at[didx_v.at[j]], add=True)
            if j + 1 < K:
                desc = ndesc
        return carry

    lax.fori_loop(0, NB, block_body, 0)

    plsc.subcore_barrier()

    pltpu.sync_copy(acc.at[pl.ds(s * zrows, zrows)],
                    out_hbm.at[c, pl.ds(s * zrows, zrows)])


def kernel(x, edge_index, edge_label, W_dir, b_lab, W_lin, b_lin):
    table = _build_table(x, W_dir, b_lab)

    src = jnp.concatenate([edge_index[0], edge_index[1]])
    dst = jnp.concatenate([edge_index[1], edge_index[0]])
    lab = jnp.concatenate([edge_label, edge_label])
    gsrc = src * jnp.int32(L) + lab

    pad = EPAD - E2
    padz = jnp.zeros((pad,), jnp.int32)
    padn = jnp.full((pad,), N, jnp.int32)
    gsrc_full = jnp.concatenate([gsrc, padz]).reshape(NW, CPW, CHUNK)
    dst_full = jnp.concatenate([dst, padn]).reshape(NW, CPW, CHUNK)

    zeros = jnp.zeros((ACC_N, D), jnp.float32)

    partials = _sc_scatter(table, gsrc_full, dst_full, zeros)
    return _final(x, W_lin, b_lin, partials[0], partials[1])


# speedup vs baseline: 4.0756x; 1.0495x over previous
import functools

import jax
import jax.numpy as jnp
from jax import lax
from jax.experimental import pallas as pl
from jax.experimental.pallas import tpu as pltpu
from jax.experimental.pallas import tpu_sc as plsc

N = 10000
E = 320000
D = 128
L = 16

NC = 2
NS = 16
NW = NC * NS
CHUNK = 128
K = 32
E2 = 2 * E
CPW0 = 96
CPW1 = 224
TOT_CHUNKS = NS * (CPW0 + CPW1)
EPAD = TOT_CHUNKS * CHUNK
ACC_N = 10112

_BM = 1000


def _table_body(x_ref, w_ref, blab_ref, o_ref):
    xw = jnp.dot(x_ref[...], w_ref[...], preferred_element_type=jnp.float32)
    msg = xw[:, None, :] + blab_ref[...][None, :, :]
    o_ref[...] = msg.reshape(_BM * L, D)


def _build_table(x, w_dir, b_lab):
    return pl.pallas_call(
        _table_body,
        grid=(N // _BM,),
        in_specs=[pl.BlockSpec((_BM, D), lambda i: (i, 0)),
                  pl.BlockSpec((D, D), lambda i: (0, 0)),
                  pl.BlockSpec((L, D), lambda i: (0, 0))],
        out_specs=pl.BlockSpec((_BM * L, D), lambda i: (i, 0)),
        out_shape=jax.ShapeDtypeStruct((N * L, D), jnp.float32),
    )(x, w_dir, b_lab)


def _final_body(x_ref, wl_ref, b_ref, p0_ref, p1_ref, o_ref):
    xl = lax.dot_general(x_ref[...], wl_ref[...],
                         (((1,), (1,)), ((), ())),
                         preferred_element_type=jnp.float32)
    o_ref[...] = jnp.maximum(xl + b_ref[...] + p0_ref[...] + p1_ref[...], 0.0)


def _final(x, w_lin, b_lin, p0, p1):
    return pl.pallas_call(
        _final_body,
        grid=(N // _BM,),
        in_specs=[pl.BlockSpec((_BM, D), lambda i: (i, 0)),
                  pl.BlockSpec((D, D), lambda i: (0, 0)),
                  pl.BlockSpec((1, D), lambda i: (0, 0)),
                  pl.BlockSpec((_BM, D), lambda i: (i, 0)),
                  pl.BlockSpec((_BM, D), lambda i: (i, 0))],
        out_specs=pl.BlockSpec((_BM, D), lambda i: (i, 0)),
        out_shape=jax.ShapeDtypeStruct((N, D), jnp.float32),
    )(x, w_lin, b_lin.reshape(1, D), p0, p1)


_sc_mesh = plsc.VectorSubcoreMesh(core_axis_name="c", subcore_axis_name="s")


@functools.partial(
    pl.kernel,
    out_type=jax.ShapeDtypeStruct((NC, ACC_N, D), jnp.float32),
    mesh=_sc_mesh,
    scratch_types=[
        pltpu.VMEM((K, CHUNK), jnp.int32),
        pltpu.VMEM((K, CHUNK), jnp.int32),
        pltpu.VMEM((CHUNK, D), jnp.float32),
        pltpu.VMEM((CHUNK, D), jnp.float32),
        pltpu.VMEM_SHARED((ACC_N, D), jnp.float32),
        pltpu.SemaphoreType.DMA,
        pltpu.SemaphoreType.DMA,
    ],
)
def _sc_scatter(table_hbm, gsrc_hbm, dst_hbm, zeros_hbm, out_hbm,
                gidx_v, didx_v, rows_a, rows_b, acc, sga, sgb):
    c = lax.axis_index("c")
    s = lax.axis_index("s")

    zrows = ACC_N // NS
    pltpu.sync_copy(zeros_hbm.at[pl.ds(s * zrows, zrows)],
                    acc.at[pl.ds(s * zrows, zrows)])
    plsc.subcore_barrier()

    rows = (rows_a, rows_b)
    g_sems = (sga, sgb)

    my_cpw = jnp.where(c == 0, CPW0, CPW1)
    my_nb = my_cpw // K
    base = c * (NS * CPW0) + s * my_cpw

    def block_body(kb, carry):
        pltpu.sync_copy(gsrc_hbm.at[pl.ds(base + kb * K, K)], gidx_v)
        pltpu.sync_copy(dst_hbm.at[pl.ds(base + kb * K, K)], didx_v)

        desc = pltpu.async_copy(table_hbm.at[gidx_v.at[0]], rows[0],
                                g_sems[0])
        for j in range(K):
            p = j % 2
            if j + 1 < K:
                ndesc = pltpu.async_copy(
                    table_hbm.at[gidx_v.at[j + 1]], rows[1 - p],
                    g_sems[1 - p])
            desc.wait()
            pltpu.sync_copy(rows[p], acc.at[didx_v.at[j]], add=True)
            if j + 1 < K:
                desc = ndesc
        return carry

    lax.fori_loop(0, my_nb, block_body, 0)

    plsc.subcore_barrier()

    pltpu.sync_copy(acc.at[pl.ds(s * zrows, zrows)],
                    out_hbm.at[c, pl.ds(s * zrows, zrows)])


def kernel(x, edge_index, edge_label, W_dir, b_lab, W_lin, b_lin):
    table = _build_table(x, W_dir, b_lab)

    src = jnp.concatenate([edge_index[0], edge_index[1]])
    dst = jnp.concatenate([edge_index[1], edge_index[0]])
    lab = jnp.concatenate([edge_label, edge_label])
    gsrc = src * jnp.int32(L) + lab

    pad = EPAD - E2
    padz = jnp.zeros((pad,), jnp.int32)
    padn = jnp.full((pad,), N, jnp.int32)
    gsrc_full = jnp.concatenate([gsrc, padz]).reshape(TOT_CHUNKS, CHUNK)
    dst_full = jnp.concatenate([dst, padn]).reshape(TOT_CHUNKS, CHUNK)

    zeros = jnp.zeros((ACC_N, D), jnp.float32)

    partials = _sc_scatter(table, gsrc_full, dst_full, zeros)
    return _final(x, W_lin, b_lin, partials[0], partials[1])


# speedup vs baseline: 4.3110x; 1.0578x over previous
import functools

import jax
import jax.numpy as jnp
from jax import lax
from jax.experimental import pallas as pl
from jax.experimental.pallas import tpu as pltpu
from jax.experimental.pallas import tpu_sc as plsc

N = 10000
E = 320000
D = 128
L = 16

NC = 2
NS = 16
NW = NC * NS
CHUNK = 128
K = 32
E2 = 2 * E
CPW0 = 224
CPW1 = 96
TOT_CHUNKS = NS * (CPW0 + CPW1)
EPAD = TOT_CHUNKS * CHUNK
ACC_N = 10112

_BM = 1000


def _table_body(x_ref, w_ref, blab_ref, o_ref):
    xw = jnp.dot(x_ref[...], w_ref[...], preferred_element_type=jnp.float32)
    msg = xw[:, None, :] + blab_ref[...][None, :, :]
    o_ref[...] = msg.reshape(_BM * L, D)


def _build_table(x, w_dir, b_lab):
    return pl.pallas_call(
        _table_body,
        grid=(N // _BM,),
        in_specs=[pl.BlockSpec((_BM, D), lambda i: (i, 0)),
                  pl.BlockSpec((D, D), lambda i: (0, 0)),
                  pl.BlockSpec((L, D), lambda i: (0, 0))],
        out_specs=pl.BlockSpec((_BM * L, D), lambda i: (i, 0)),
        out_shape=jax.ShapeDtypeStruct((N * L, D), jnp.float32),
    )(x, w_dir, b_lab)


def _final_body(x_ref, wl_ref, b_ref, p0_ref, p1_ref, o_ref):
    xl = lax.dot_general(x_ref[...], wl_ref[...],
                         (((1,), (1,)), ((), ())),
                         preferred_element_type=jnp.float32)
    o_ref[...] = jnp.maximum(xl + b_ref[...] + p0_ref[...] + p1_ref[...], 0.0)


def _final(x, w_lin, b_lin, p0, p1):
    return pl.pallas_call(
        _final_body,
        grid=(N // _BM,),
        in_specs=[pl.BlockSpec((_BM, D), lambda i: (i, 0)),
                  pl.BlockSpec((D, D), lambda i: (0, 0)),
                  pl.BlockSpec((1, D), lambda i: (0, 0)),
                  pl.BlockSpec((_BM, D), lambda i: (i, 0)),
                  pl.BlockSpec((_BM, D), lambda i: (i, 0))],
        out_specs=pl.BlockSpec((_BM, D), lambda i: (i, 0)),
        out_shape=jax.ShapeDtypeStruct((N, D), jnp.float32),
    )(x, w_lin, b_lin.reshape(1, D), p0, p1)


_sc_mesh = plsc.VectorSubcoreMesh(core_axis_name="c", subcore_axis_name="s")


@functools.partial(
    pl.kernel,
    out_type=jax.ShapeDtypeStruct((NC, ACC_N, D), jnp.float32),
    mesh=_sc_mesh,
    scratch_types=[
        pltpu.VMEM((K, CHUNK), jnp.int32),
        pltpu.VMEM((K, CHUNK), jnp.int32),
        pltpu.VMEM((CHUNK, D), jnp.float32),
        pltpu.VMEM((CHUNK, D), jnp.float32),
        pltpu.VMEM_SHARED((ACC_N, D), jnp.float32),
        pltpu.SemaphoreType.DMA,
        pltpu.SemaphoreType.DMA,
    ],
)
def _sc_scatter(table_hbm, gsrc_hbm, dst_hbm, zeros_hbm, out_hbm,
                gidx_v, didx_v, rows_a, rows_b, acc, sga, sgb):
    c = lax.axis_index("c")
    s = lax.axis_index("s")

    zrows = ACC_N // NS
    pltpu.sync_copy(zeros_hbm.at[pl.ds(s * zrows, zrows)],
                    acc.at[pl.ds(s * zrows, zrows)])
    plsc.subcore_barrier()

    rows = (rows_a, rows_b)
    g_sems = (sga, sgb)

    my_cpw = jnp.where(c == 0, CPW0, CPW1)
    my_nb = my_cpw // K
    base = c * (NS * CPW0) + s * my_cpw

    def block_body(kb, carry):
        pltpu.sync_copy(gsrc_hbm.at[pl.ds(base + kb * K, K)], gidx_v)
        pltpu.sync_copy(dst_hbm.at[pl.ds(base + kb * K, K)], didx_v)

        desc = pltpu.async_copy(table_hbm.at[gidx_v.at[0]], rows[0],
                                g_sems[0])
        for j in range(K):
            p = j % 2
            if j + 1 < K:
                ndesc = pltpu.async_copy(
                    table_hbm.at[gidx_v.at[j + 1]], rows[1 - p],
                    g_sems[1 - p])
            desc.wait()
            pltpu.sync_copy(rows[p], acc.at[didx_v.at[j]], add=True)
            if j + 1 < K:
                desc = ndesc
        return carry

    lax.fori_loop(0, my_nb, block_body, 0)

    plsc.subcore_barrier()

    pltpu.sync_copy(acc.at[pl.ds(s * zrows, zrows)],
                    out_hbm.at[c, pl.ds(s * zrows, zrows)])


def kernel(x, edge_index, edge_label, W_dir, b_lab, W_lin, b_lin):
    table = _build_table(x, W_dir, b_lab)

    src = jnp.concatenate([edge_index[0], edge_index[1]])
    dst = jnp.concatenate([edge_index[1], edge_index[0]])
    lab = jnp.concatenate([edge_label, edge_label])
    gsrc = src * jnp.int32(L) + lab

    pad = EPAD - E2
    padz = jnp.zeros((pad,), jnp.int32)
    padn = jnp.full((pad,), N, jnp.int32)
    gsrc_full = jnp.concatenate([gsrc, padz]).reshape(TOT_CHUNKS, CHUNK)
    dst_full = jnp.concatenate([dst, padn]).reshape(TOT_CHUNKS, CHUNK)

    zeros = jnp.zeros((ACC_N, D), jnp.float32)

    partials = _sc_scatter(table, gsrc_full, dst_full, zeros)
    return _final(x, W_lin, b_lin, partials[0], partials[1])


# speedup vs baseline: 4.3375x; 1.0061x over previous
import functools

import jax
import jax.numpy as jnp
from jax import lax
from jax.experimental import pallas as pl
from jax.experimental.pallas import tpu as pltpu
from jax.experimental.pallas import tpu_sc as plsc

N = 10000
E = 320000
D = 128
L = 16

NC = 2
NS = 16
NW = NC * NS
CHUNK = 128
K = 32
E2 = 2 * E
CPW0 = 256
CPW1 = 64
TOT_CHUNKS = NS * (CPW0 + CPW1)
EPAD = TOT_CHUNKS * CHUNK
ACC_N = 10112

_BM = 1000


def _table_body(x_ref, w_ref, blab_ref, o_ref):
    xw = jnp.dot(x_ref[...], w_ref[...], preferred_element_type=jnp.float32)
    msg = xw[:, None, :] + blab_ref[...][None, :, :]
    o_ref[...] = msg.reshape(_BM * L, D)


def _build_table(x, w_dir, b_lab):
    return pl.pallas_call(
        _table_body,
        grid=(N // _BM,),
        in_specs=[pl.BlockSpec((_BM, D), lambda i: (i, 0)),
                  pl.BlockSpec((D, D), lambda i: (0, 0)),
                  pl.BlockSpec((L, D), lambda i: (0, 0))],
        out_specs=pl.BlockSpec((_BM * L, D), lambda i: (i, 0)),
        out_shape=jax.ShapeDtypeStruct((N * L, D), jnp.float32),
    )(x, w_dir, b_lab)


def _final_body(x_ref, wl_ref, b_ref, p0_ref, p1_ref, o_ref):
    xl = lax.dot_general(x_ref[...], wl_ref[...],
                         (((1,), (1,)), ((), ())),
                         preferred_element_type=jnp.float32)
    o_ref[...] = jnp.maximum(xl + b_ref[...] + p0_ref[...] + p1_ref[...], 0.0)


def _final(x, w_lin, b_lin, p0, p1):
    return pl.pallas_call(
        _final_body,
        grid=(N // _BM,),
        in_specs=[pl.BlockSpec((_BM, D), lambda i: (i, 0)),
                  pl.BlockSpec((D, D), lambda i: (0, 0)),
                  pl.BlockSpec((1, D), lambda i: (0, 0)),
                  pl.BlockSpec((_BM, D), lambda i: (i, 0)),
                  pl.BlockSpec((_BM, D), lambda i: (i, 0))],
        out_specs=pl.BlockSpec((_BM, D), lambda i: (i, 0)),
        out_shape=jax.ShapeDtypeStruct((N, D), jnp.float32),
    )(x, w_lin, b_lin.reshape(1, D), p0, p1)


_sc_mesh = plsc.VectorSubcoreMesh(core_axis_name="c", subcore_axis_name="s")


@functools.partial(
    pl.kernel,
    out_type=jax.ShapeDtypeStruct((NC, ACC_N, D), jnp.float32),
    mesh=_sc_mesh,
    scratch_types=[
        pltpu.VMEM((K, CHUNK), jnp.int32),
        pltpu.VMEM((K, CHUNK), jnp.int32),
        pltpu.VMEM((CHUNK, D), jnp.float32),
        pltpu.VMEM((CHUNK, D), jnp.float32),
        pltpu.VMEM_SHARED((ACC_N, D), jnp.float32),
        pltpu.SemaphoreType.DMA,
        pltpu.SemaphoreType.DMA,
    ],
)
def _sc_scatter(table_hbm, gsrc_hbm, dst_hbm, zeros_hbm, out_hbm,
                gidx_v, didx_v, rows_a, rows_b, acc, sga, sgb):
    c = lax.axis_index("c")
    s = lax.axis_index("s")

    zrows = ACC_N // NS
    pltpu.sync_copy(zeros_hbm.at[pl.ds(s * zrows, zrows)],
                    acc.at[pl.ds(s * zrows, zrows)])
    plsc.subcore_barrier()

    rows = (rows_a, rows_b)
    g_sems = (sga, sgb)

    my_cpw = jnp.where(c == 0, CPW0, CPW1)
    my_nb = my_cpw // K
    base = c * (NS * CPW0) + s * my_cpw

    def block_body(kb, carry):
        pltpu.sync_copy(gsrc_hbm.at[pl.ds(base + kb * K, K)], gidx_v)
        pltpu.sync_copy(dst_hbm.at[pl.ds(base + kb * K, K)], didx_v)

        desc = pltpu.async_copy(table_hbm.at[gidx_v.at[0]], rows[0],
                                g_sems[0])
        for j in range(K):
            p = j % 2
            if j + 1 < K:
                ndesc = pltpu.async_copy(
                    table_hbm.at[gidx_v.at[j + 1]], rows[1 - p],
                    g_sems[1 - p])
            desc.wait()
            pltpu.sync_copy(rows[p], acc.at[didx_v.at[j]], add=True)
            if j + 1 < K:
                desc = ndesc
        return carry

    lax.fori_loop(0, my_nb, block_body, 0)

    plsc.subcore_barrier()

    pltpu.sync_copy(acc.at[pl.ds(s * zrows, zrows)],
                    out_hbm.at[c, pl.ds(s * zrows, zrows)])


def kernel(x, edge_index, edge_label, W_dir, b_lab, W_lin, b_lin):
    table = _build_table(x, W_dir, b_lab)

    src = jnp.concatenate([edge_index[0], edge_index[1]])
    dst = jnp.concatenate([edge_index[1], edge_index[0]])
    lab = jnp.concatenate([edge_label, edge_label])
    gsrc = src * jnp.int32(L) + lab

    pad = EPAD - E2
    padz = jnp.zeros((pad,), jnp.int32)
    padn = jnp.full((pad,), N, jnp.int32)
    gsrc_full = jnp.concatenate([gsrc, padz]).reshape(TOT_CHUNKS, CHUNK)
    dst_full = jnp.concatenate([dst, padn]).reshape(TOT_CHUNKS, CHUNK)

    zeros = jnp.zeros((ACC_N, D), jnp.float32)

    partials = _sc_scatter(table, gsrc_full, dst_full, zeros)
    return _final(x, W_lin, b_lin, partials[0], partials[1])
